# baseline (speedup 1.0000x reference)
import functools

import jax
import jax.numpy as jnp
from jax import lax
from jax.experimental import pallas as pl
from jax.experimental.pallas import tpu as pltpu
from jax.experimental.pallas import tpu_sc as plsc

N = 10000
D = 128
EMB = 256
H = 128
E = 320000

TILES = 16
CHUNK = 88
N_CHUNKS = 228
EDGES_PER_TILE = N_CHUNKS * CHUNK
E_PAD = EDGES_PER_TILE * TILES
STRIPE = 632
N_PAD = STRIPE * TILES
NBUF = 4
IBUF = 8
IDX_ROWS_PER_CORE = TILES * N_CHUNKS * 2

_F32 = jnp.float32


def _sc_agg_body(h_hbm, idx_hbm, zeros_hbm, out_hbm,
                 idx_v, rows, acc, si, sg, ss):
    c = lax.axis_index("c")
    s = lax.axis_index("s")

    def idx_start(g, bi):
        row0 = c * IDX_ROWS_PER_CORE + (s * N_CHUNKS + g) * 2
        pltpu.async_copy(idx_hbm.at[pl.ds(row0, 2)], idx_v.at[bi], si.at[bi])

    def idx_wait(bi):
        pltpu.make_async_copy(idx_hbm.at[pl.ds(0, 2)], idx_v.at[bi],
                              si.at[bi]).wait()

    def gather_start(b, bi):
        pltpu.async_copy(h_hbm.at[idx_v.at[bi, 0]], rows.at[b], sg.at[b])

    def gather_wait(b):
        pltpu.make_async_copy(h_hbm.at[idx_v.at[0, 0]], rows.at[b],
                              sg.at[b]).wait()

    def scatter_start(b, bi):
        pltpu.async_copy(rows.at[b], acc.at[idx_v.at[bi, 1]], ss.at[b],
                         add=True)

    def scatter_wait(b):
        pltpu.make_async_copy(rows.at[b], acc.at[idx_v.at[0, 1]],
                              ss.at[b]).wait()

    pltpu.sync_copy(zeros_hbm, acc.at[pl.ds(s * STRIPE, STRIPE)])
    plsc.subcore_barrier()

    for k in range(4):
        idx_start(k, k)
    for k in range(3):
        idx_wait(k)
        gather_start(k, k)
    gather_wait(0)
    scatter_start(0, 0)
    idx_start(4, 4)
    idx_wait(3)
    gather_start(3, 3)

    def steady(g, carry):
        b = lax.rem(g, NBUF)
        bp = lax.rem(g - 1, NBUF)
        bf = lax.rem(g + 3, NBUF)
        bi = lax.rem(g, IBUF)
        bif = lax.rem(g + 3, IBUF)
        bin_ = lax.rem(g + 4, IBUF)
        gather_wait(b)
        scatter_start(b, bi)
        scatter_wait(bp)
        idx_start(g + 4, bin_)
        idx_wait(bif)
        gather_start(bf, bif)
        return carry

    lax.fori_loop(1, N_CHUNKS - 4, steady, 0)

    g = N_CHUNKS - 4
    gather_wait(g % NBUF)
    scatter_start(g % NBUF, g % IBUF)
    scatter_wait((g - 1) % NBUF)
    idx_wait((g + 3) % IBUF)
    gather_start((g + 3) % NBUF, (g + 3) % IBUF)
    for g in range(N_CHUNKS - 3, N_CHUNKS):
        gather_wait(g % NBUF)
        scatter_start(g % NBUF, g % IBUF)
        scatter_wait((g - 1) % NBUF)
    scatter_wait((N_CHUNKS - 1) % NBUF)

    plsc.subcore_barrier()
    pltpu.sync_copy(acc.at[pl.ds(s * STRIPE, STRIPE)],
                    out_hbm.at[pl.ds(c * N_PAD + s * STRIPE, STRIPE)])


_sc_agg = pl.kernel(
    _sc_agg_body,
    mesh=plsc.VectorSubcoreMesh(core_axis_name="c", subcore_axis_name="s"),
    out_type=jax.ShapeDtypeStruct((2 * N_PAD, H), _F32),
    scratch_types=[
        pltpu.VMEM((IBUF, 2, CHUNK), jnp.int32),
        pltpu.VMEM((NBUF, CHUNK, H), _F32),
        pltpu.VMEM_SHARED((N_PAD, H), _F32),
        pltpu.SemaphoreType.DMA((IBUF,)),
        pltpu.SemaphoreType.DMA((NBUF,)),
        pltpu.SemaphoreType.DMA((NBUF,)),
    ],
)


_ROWS = 1000


def _proj_body(x_ref, w_ref, b_ref, out_ref):
    res = jnp.dot(x_ref[...], w_ref[...], preferred_element_type=_F32)
    res = res + b_ref[...]
    out_ref[0] = res[:, :H]
    out_ref[1] = res[:, H:]


_proj = pl.pallas_call(
    _proj_body,
    grid=(N // _ROWS,),
    in_specs=[
        pl.BlockSpec((_ROWS, D), lambda i: (i, 0)),
        pl.BlockSpec((D, EMB), lambda i: (0, 0)),
        pl.BlockSpec((1, EMB), lambda i: (0, 0)),
    ],
    out_specs=pl.BlockSpec((2, _ROWS, H), lambda i: (0, i, 0)),
    out_shape=jax.ShapeDtypeStruct((2, N, H), _F32),
)


def _part_body(h_ref, wr_ref, b_ref, out_ref):
    res = (jnp.dot(h_ref[0], wr_ref[:H, :], preferred_element_type=_F32)
           + jnp.dot(h_ref[1], wr_ref[H:, :], preferred_element_type=_F32)
           + b_ref[...])
    out_ref[0] = res[:, :H]
    out_ref[1] = res[:, H:]


_part = pl.pallas_call(
    _part_body,
    grid=(N // _ROWS,),
    in_specs=[
        pl.BlockSpec((2, _ROWS, H), lambda i: (0, i, 0)),
        pl.BlockSpec((EMB, EMB), lambda i: (0, 0)),
        pl.BlockSpec((1, EMB), lambda i: (0, 0)),
    ],
    out_specs=pl.BlockSpec((2, _ROWS, H), lambda i: (0, i, 0)),
    out_shape=jax.ShapeDtypeStruct((2, N, H), _F32),
)


def _fin_body(p_ref, a_ref, wn_ref, out_ref, *, relu, split):
    res = (jnp.concatenate([p_ref[0], p_ref[1]], axis=1)
           + jnp.dot(a_ref[0], wn_ref[:H, :], preferred_element_type=_F32)
           + jnp.dot(a_ref[1], wn_ref[H:, :], preferred_element_type=_F32))
    if relu:
        res = jnp.maximum(res, 0.0)
    if split:
        out_ref[0] = res[:, :H]
        out_ref[1] = res[:, H:]
    else:
        out_ref[...] = res


def _make_fin(relu, split):
    if split:
        out_spec = pl.BlockSpec((2, _ROWS, H), lambda i: (0, i, 0))
        out_shape = jax.ShapeDtypeStruct((2, N, H), _F32)
    else:
        out_spec = pl.BlockSpec((_ROWS, EMB), lambda i: (i, 0))
        out_shape = jax.ShapeDtypeStruct((N, EMB), _F32)
    return pl.pallas_call(
        functools.partial(_fin_body, relu=relu, split=split),
        grid=(N // _ROWS,),
        in_specs=[
            pl.BlockSpec((2, _ROWS, H), lambda i: (0, i, 0)),
            pl.BlockSpec((2, _ROWS, H), lambda i: (0, i, 0)),
            pl.BlockSpec((EMB, EMB), lambda i: (0, 0)),
        ],
        out_specs=out_spec,
        out_shape=out_shape,
    )


_fin_mid = _make_fin(relu=True, split=True)
_fin_last = _make_fin(relu=False, split=False)


def kernel(x, edge_index, edge_attr, Wx, bx,
           Wr0, Wn0, b0, Wr1, Wn1, b1, Wr2, Wn2, b2):
    del edge_attr
    src = edge_index[0].astype(jnp.int32)
    dst = edge_index[1].astype(jnp.int32)

    pad = E_PAD - E
    src_p = jnp.concatenate([src, jnp.zeros((pad,), jnp.int32)])
    dst_p = jnp.concatenate([dst, jnp.full((pad,), N_PAD - 1, jnp.int32)])
    s3 = src_p.reshape(TILES * N_CHUNKS, CHUNK)
    d3 = dst_p.reshape(TILES * N_CHUNKS, CHUNK)
    idx = jnp.concatenate([jnp.stack([s3, d3], axis=1),
                           jnp.stack([s3 + N, d3], axis=1)]).reshape(-1, CHUNK)
    zeros = jnp.zeros((STRIPE, H), _F32)

    h2 = _proj(x, Wx, bx.reshape(1, EMB))
    for Wr, Wn, b, fin_fn in (
        (Wr0, Wn0, b0, _fin_mid),
        (Wr1, Wn1, b1, _fin_mid),
        (Wr2, Wn2, b2, _fin_last),
    ):
        agg_flat = _sc_agg(h2.reshape(2 * N, H), idx, zeros)
        part = _part(h2, Wr, b.reshape(1, EMB))
        agg2 = agg_flat.reshape(2, N_PAD, H)
        h2 = fin_fn(part, agg2, Wn)
    return h2


# speedup vs baseline: 1.1192x; 1.1192x over previous
import functools

import jax
import jax.numpy as jnp
from jax import lax
from jax.experimental import pallas as pl
from jax.experimental.pallas import tpu as pltpu
from jax.experimental.pallas import tpu_sc as plsc

N = 10000
D = 128
EMB = 256
H = 128
E = 320000

TILES = 16
CHUNK = 120
N_CHUNKS = 167
EDGES_PER_TILE = N_CHUNKS * CHUNK
E_PAD = EDGES_PER_TILE * TILES
STRIPE = 632
N_PAD = STRIPE * TILES
NBUF = 3
IBUF = 8
IDX_ROWS_PER_CORE = TILES * N_CHUNKS * 2

_F32 = jnp.float32
_I32 = jnp.int32


def _sc_agg_body(h_hbm, idx_hbm, zeros_hbm, out_hbm,
                 idx_v, rows, acc, si, sg, ss):
    c = lax.axis_index("c")
    s = lax.axis_index("s")

    def idx_start(g, bi):
        row0 = c * IDX_ROWS_PER_CORE + (s * N_CHUNKS + g) * 2
        pltpu.async_copy(idx_hbm.at[pl.ds(row0, 2)], idx_v.at[bi], si.at[bi])

    def idx_wait(bi):
        pltpu.make_async_copy(idx_hbm.at[pl.ds(0, 2)], idx_v.at[bi],
                              si.at[bi]).wait()

    def gather_start(b, bi):
        pltpu.async_copy(h_hbm.at[idx_v.at[bi, 0]], rows.at[b], sg.at[b])

    def gather_wait(b):
        pltpu.make_async_copy(h_hbm.at[idx_v.at[0, 0]], rows.at[b],
                              sg.at[b]).wait()

    def scatter_start(b, bi):
        pltpu.async_copy(rows.at[b], acc.at[idx_v.at[bi, 1]], ss.at[b],
                         add=True)

    def scatter_wait(b):
        pltpu.make_async_copy(rows.at[b], acc.at[idx_v.at[0, 1]],
                              ss.at[b]).wait()

    pltpu.sync_copy(zeros_hbm, acc.at[pl.ds(s * STRIPE, STRIPE)])
    plsc.subcore_barrier()

    for k in range(3):
        idx_start(k, k)
    for k in range(2):
        idx_wait(k)
        gather_start(k, k)

    def round_(g, carry):
        b = lax.rem(g, NBUF)
        bp = lax.rem(g - 1, NBUF)
        bf = lax.rem(g + 2, NBUF)
        bi = lax.rem(g, IBUF)
        bif = lax.rem(g + 2, IBUF)
        bin_ = lax.rem(g + 3, IBUF)
        gather_wait(b)
        scatter_start(b, bi)

        @pl.when(g >= 1)
        def _():
            scatter_wait(bp)

        @pl.when(g + 3 < N_CHUNKS)
        def _():
            idx_start(g + 3, bin_)

        @pl.when(g + 2 < N_CHUNKS)
        def _():
            idx_wait(bif)
            gather_start(bf, bif)
        return carry

    lax.fori_loop(0, N_CHUNKS, round_, 0)
    scatter_wait((N_CHUNKS - 1) % NBUF)

    plsc.subcore_barrier()
    pltpu.sync_copy(acc.at[pl.ds(s * STRIPE, STRIPE)],
                    out_hbm.at[pl.ds(c * N_PAD + s * STRIPE, STRIPE)])


_sc_agg = pl.kernel(
    _sc_agg_body,
    mesh=plsc.VectorSubcoreMesh(core_axis_name="c", subcore_axis_name="s"),
    out_type=jax.ShapeDtypeStruct((2 * N_PAD, H), _F32),
    scratch_types=[
        pltpu.VMEM((IBUF, 2, CHUNK), _I32),
        pltpu.VMEM((NBUF, CHUNK, H), _F32),
        pltpu.VMEM_SHARED((N_PAD, H), _F32),
        pltpu.SemaphoreType.DMA((IBUF,)),
        pltpu.SemaphoreType.DMA((NBUF,)),
        pltpu.SemaphoreType.DMA((NBUF,)),
    ],
)


_ROWS = 1000


def _proj_body(x_ref, w_ref, b_ref, out_ref):
    res = jnp.dot(x_ref[...], w_ref[...], preferred_element_type=_F32)
    res = res + b_ref[...]
    out_ref[0] = res[:, :H]
    out_ref[1] = res[:, H:]


_proj = pl.pallas_call(
    _proj_body,
    grid=(N // _ROWS,),
    in_specs=[
        pl.BlockSpec((_ROWS, D), lambda i: (i, 0)),
        pl.BlockSpec((D, EMB), lambda i: (0, 0)),
        pl.BlockSpec((1, EMB), lambda i: (0, 0)),
    ],
    out_specs=pl.BlockSpec((2, _ROWS, H), lambda i: (0, i, 0)),
    out_shape=jax.ShapeDtypeStruct((2, N, H), _F32),
)


def _part_body(h_ref, wr_ref, b_ref, out_ref):
    res = (jnp.dot(h_ref[0], wr_ref[:H, :], preferred_element_type=_F32)
           + jnp.dot(h_ref[1], wr_ref[H:, :], preferred_element_type=_F32)
           + b_ref[...])
    out_ref[0] = res[:, :H]
    out_ref[1] = res[:, H:]


_part = pl.pallas_call(
    _part_body,
    grid=(N // _ROWS,),
    in_specs=[
        pl.BlockSpec((2, _ROWS, H), lambda i: (0, i, 0)),
        pl.BlockSpec((EMB, EMB), lambda i: (0, 0)),
        pl.BlockSpec((1, EMB), lambda i: (0, 0)),
    ],
    out_specs=pl.BlockSpec((2, _ROWS, H), lambda i: (0, i, 0)),
    out_shape=jax.ShapeDtypeStruct((2, N, H), _F32),
)


def _fin_body(p_ref, a_ref, wn_ref, out_ref, *, relu, split):
    res = (jnp.concatenate([p_ref[0], p_ref[1]], axis=1)
           + jnp.dot(a_ref[0], wn_ref[:H, :], preferred_element_type=_F32)
           + jnp.dot(a_ref[1], wn_ref[H:, :], preferred_element_type=_F32))
    if relu:
        res = jnp.maximum(res, 0.0)
    if split:
        out_ref[0] = res[:, :H]
        out_ref[1] = res[:, H:]
    else:
        out_ref[...] = res


def _make_fin(relu, split):
    if split:
        out_spec = pl.BlockSpec((2, _ROWS, H), lambda i: (0, i, 0))
        out_shape = jax.ShapeDtypeStruct((2, N, H), _F32)
    else:
        out_spec = pl.BlockSpec((_ROWS, EMB), lambda i: (i, 0))
        out_shape = jax.ShapeDtypeStruct((N, EMB), _F32)
    return pl.pallas_call(
        functools.partial(_fin_body, relu=relu, split=split),
        grid=(N // _ROWS,),
        in_specs=[
            pl.BlockSpec((2, _ROWS, H), lambda i: (0, i, 0)),
            pl.BlockSpec((2, _ROWS, H), lambda i: (0, i, 0)),
            pl.BlockSpec((EMB, EMB), lambda i: (0, 0)),
        ],
        out_specs=out_spec,
        out_shape=out_shape,
    )


_fin_mid = _make_fin(relu=True, split=True)
_fin_last = _make_fin(relu=False, split=False)


def kernel(x, edge_index, edge_attr, Wx, bx,
           Wr0, Wn0, b0, Wr1, Wn1, b1, Wr2, Wn2, b2):
    del edge_attr
    src = edge_index[0].astype(_I32)
    dst = edge_index[1].astype(_I32)

    pad = E_PAD - E
    src_p = jnp.concatenate([src, jnp.zeros((pad,), _I32)])
    dst_p = jnp.concatenate([dst, jnp.full((pad,), N_PAD - 1, _I32)])
    s3 = src_p.reshape(TILES * N_CHUNKS, CHUNK)
    d3 = dst_p.reshape(TILES * N_CHUNKS, CHUNK)
    idx = jnp.concatenate([jnp.stack([s3, d3], axis=1),
                           jnp.stack([s3 + N, d3], axis=1)]).reshape(-1, CHUNK)
    zeros = jnp.zeros((STRIPE, H), _F32)

    h2 = _proj(x, Wx, bx.reshape(1, EMB))
    for Wr, Wn, b, fin_fn in (
        (Wr0, Wn0, b0, _fin_mid),
        (Wr1, Wn1, b1, _fin_mid),
        (Wr2, Wn2, b2, _fin_last),
    ):
        agg_flat = _sc_agg(h2.reshape(2 * N, H), idx, zeros)
        part = _part(h2, Wr, b.reshape(1, EMB))
        agg2 = agg_flat.reshape(2, N_PAD, H)
        h2 = fin_fn(part, agg2, Wn)
    return h2


# speedup vs baseline: 1.1223x; 1.0028x over previous
import functools

import jax
import jax.numpy as jnp
from jax import lax
from jax.experimental import pallas as pl
from jax.experimental.pallas import tpu as pltpu
from jax.experimental.pallas import tpu_sc as plsc

N = 10000
D = 128
EMB = 256
H = 128
E = 320000

TILES = 16
CHUNK = 120
N_CHUNKS = 167
EDGES_PER_TILE = N_CHUNKS * CHUNK
E_PAD = EDGES_PER_TILE * TILES
STRIPE = 632
N_PAD = STRIPE * TILES
NBUF = 3
IBUF = 8
IDX_ROWS_PER_CORE = TILES * N_CHUNKS * 2

_F32 = jnp.float32
_I32 = jnp.int32


def _sc_agg_body(h_hbm, idx_hbm, zeros_hbm, out_hbm,
                 idx_v, rows, acc, si, sg, ss):
    c = lax.axis_index("c")
    s = lax.axis_index("s")

    def idx_start(g, bi):
        row0 = c * IDX_ROWS_PER_CORE + (s * N_CHUNKS + g) * 2
        pltpu.async_copy(idx_hbm.at[pl.ds(row0, 2)], idx_v.at[bi], si.at[bi])

    def idx_wait(bi):
        pltpu.make_async_copy(idx_hbm.at[pl.ds(0, 2)], idx_v.at[bi],
                              si.at[bi]).wait()

    def gather_start(b, bi):
        pltpu.async_copy(h_hbm.at[idx_v.at[bi, 0]], rows.at[b], sg.at[b])

    def gather_wait(b):
        pltpu.make_async_copy(h_hbm.at[idx_v.at[0, 0]], rows.at[b],
                              sg.at[b]).wait()

    def scatter_start(b, bi):
        pltpu.async_copy(rows.at[b], acc.at[idx_v.at[bi, 1]], ss.at[b],
                         add=True)

    def scatter_wait(b):
        pltpu.make_async_copy(rows.at[b], acc.at[idx_v.at[0, 1]],
                              ss.at[b]).wait()

    zcp = pltpu.async_copy(zeros_hbm, acc.at[pl.ds(s * STRIPE, STRIPE)],
                           ss.at[0])
    for k in range(3):
        idx_start(k, k)
    for k in range(2):
        idx_wait(k)
        gather_start(k, k)
    zcp.wait()
    plsc.subcore_barrier()


    def round_(g, carry):
        b = lax.rem(g, NBUF)
        bp = lax.rem(g - 1, NBUF)
        bf = lax.rem(g + 2, NBUF)
        bi = lax.rem(g, IBUF)
        bif = lax.rem(g + 2, IBUF)
        bin_ = lax.rem(g + 3, IBUF)
        gather_wait(b)
        scatter_start(b, bi)

        @pl.when(g >= 1)
        def _():
            scatter_wait(bp)

        @pl.when(g + 3 < N_CHUNKS)
        def _():
            idx_start(g + 3, bin_)

        @pl.when(g + 2 < N_CHUNKS)
        def _():
            idx_wait(bif)
            gather_start(bf, bif)
        return carry

    lax.fori_loop(0, N_CHUNKS, round_, 0)
    scatter_wait((N_CHUNKS - 1) % NBUF)

    plsc.subcore_barrier()
    pltpu.sync_copy(acc.at[pl.ds(s * STRIPE, STRIPE)],
                    out_hbm.at[pl.ds(c * N_PAD + s * STRIPE, STRIPE)])


_sc_agg = pl.kernel(
    _sc_agg_body,
    mesh=plsc.VectorSubcoreMesh(core_axis_name="c", subcore_axis_name="s"),
    out_type=jax.ShapeDtypeStruct((2 * N_PAD, H), _F32),
    scratch_types=[
        pltpu.VMEM((IBUF, 2, CHUNK), _I32),
        pltpu.VMEM((NBUF, CHUNK, H), _F32),
        pltpu.VMEM_SHARED((N_PAD, H), _F32),
        pltpu.SemaphoreType.DMA((IBUF,)),
        pltpu.SemaphoreType.DMA((NBUF,)),
        pltpu.SemaphoreType.DMA((NBUF,)),
    ],
)


_ROWS = 1000


def _proj_body(x_ref, w_ref, b_ref, out_ref):
    res = jnp.dot(x_ref[...], w_ref[...], preferred_element_type=_F32)
    res = res + b_ref[...]
    out_ref[0] = res[:, :H]
    out_ref[1] = res[:, H:]


_proj = pl.pallas_call(
    _proj_body,
    grid=(N // _ROWS,),
    in_specs=[
        pl.BlockSpec((_ROWS, D), lambda i: (i, 0)),
        pl.BlockSpec((D, EMB), lambda i: (0, 0)),
        pl.BlockSpec((1, EMB), lambda i: (0, 0)),
    ],
    out_specs=pl.BlockSpec((2, _ROWS, H), lambda i: (0, i, 0)),
    out_shape=jax.ShapeDtypeStruct((2, N, H), _F32),
)


def _part_body(h_ref, wr_ref, b_ref, out_ref):
    res = (jnp.dot(h_ref[0], wr_ref[:H, :], preferred_element_type=_F32)
           + jnp.dot(h_ref[1], wr_ref[H:, :], preferred_element_type=_F32)
           + b_ref[...])
    out_ref[0] = res[:, :H]
    out_ref[1] = res[:, H:]


_part = pl.pallas_call(
    _part_body,
    grid=(N // _ROWS,),
    in_specs=[
        pl.BlockSpec((2, _ROWS, H), lambda i: (0, i, 0)),
        pl.BlockSpec((EMB, EMB), lambda i: (0, 0)),
        pl.BlockSpec((1, EMB), lambda i: (0, 0)),
    ],
    out_specs=pl.BlockSpec((2, _ROWS, H), lambda i: (0, i, 0)),
    out_shape=jax.ShapeDtypeStruct((2, N, H), _F32),
)


def _fin_body(p_ref, a_ref, wn_ref, out_ref, *, relu, split):
    res = (jnp.concatenate([p_ref[0], p_ref[1]], axis=1)
           + jnp.dot(a_ref[0], wn_ref[:H, :], preferred_element_type=_F32)
           + jnp.dot(a_ref[1], wn_ref[H:, :], preferred_element_type=_F32))
    if relu:
        res = jnp.maximum(res, 0.0)
    if split:
        out_ref[0] = res[:, :H]
        out_ref[1] = res[:, H:]
    else:
        out_ref[...] = res


def _make_fin(relu, split):
    if split:
        out_spec = pl.BlockSpec((2, _ROWS, H), lambda i: (0, i, 0))
        out_shape = jax.ShapeDtypeStruct((2, N, H), _F32)
    else:
        out_spec = pl.BlockSpec((_ROWS, EMB), lambda i: (i, 0))
        out_shape = jax.ShapeDtypeStruct((N, EMB), _F32)
    return pl.pallas_call(
        functools.partial(_fin_body, relu=relu, split=split),
        grid=(N // _ROWS,),
        in_specs=[
            pl.BlockSpec((2, _ROWS, H), lambda i: (0, i, 0)),
            pl.BlockSpec((2, _ROWS, H), lambda i: (0, i, 0)),
            pl.BlockSpec((EMB, EMB), lambda i: (0, 0)),
        ],
        out_specs=out_spec,
        out_shape=out_shape,
    )


_fin_mid = _make_fin(relu=True, split=True)
_fin_last = _make_fin(relu=False, split=False)


def kernel(x, edge_index, edge_attr, Wx, bx,
           Wr0, Wn0, b0, Wr1, Wn1, b1, Wr2, Wn2, b2):
    del edge_attr
    src = edge_index[0].astype(_I32)
    dst = edge_index[1].astype(_I32)

    pad = E_PAD - E
    src_p = jnp.concatenate([src, jnp.zeros((pad,), _I32)])
    dst_p = jnp.concatenate([dst, jnp.full((pad,), N_PAD - 1, _I32)])
    s3 = src_p.reshape(TILES * N_CHUNKS, CHUNK)
    d3 = dst_p.reshape(TILES * N_CHUNKS, CHUNK)
    idx = jnp.concatenate([jnp.stack([s3, d3], axis=1),
                           jnp.stack([s3 + N, d3], axis=1)]).reshape(-1, CHUNK)
    zeros = jnp.zeros((STRIPE, H), _F32)

    h2 = _proj(x, Wx, bx.reshape(1, EMB))
    for Wr, Wn, b, fin_fn in (
        (Wr0, Wn0, b0, _fin_mid),
        (Wr1, Wn1, b1, _fin_mid),
        (Wr2, Wn2, b2, _fin_last),
    ):
        agg_flat = _sc_agg(h2.reshape(2 * N, H), idx, zeros)
        part = _part(h2, Wr, b.reshape(1, EMB))
        agg2 = agg_flat.reshape(2, N_PAD, H)
        h2 = fin_fn(part, agg2, Wn)
    return h2
